# Initial kernel scaffold; baseline (speedup 1.0000x reference)
import jax
import jax.numpy as jnp
from jax.experimental import pallas as pl


def kernel(ufea, vfea, Wu, Wv, bu, bv, attW_a, attb_a, attq_a, attW_b, attb_b, attq_b, edge_index):
    raise NotImplementedError("write your pallas kernel here")


# speedup vs baseline: 3.9729x; 3.9729x over previous
import functools

import jax
import jax.numpy as jnp
from jax import lax
from jax.experimental import pallas as pl
from jax.experimental.pallas import tpu as pltpu
from jax.experimental.pallas import tpu_sc as plsc

N_NODE = 10000
N_EDGE = 160000
FT = 256
OUT = 512
NC = 2
NS = 16
KC = 16
CW = OUT // KC
EB = 512
EPT = 10240
NB = EPT // EB
NBA = 24
E_PAD = NS * EPT
GR = 240
R_ACC = N_NODE + GR
ZR = R_ACC // NS
OR = N_NODE // NS
BM = 1000


def _mm_body(fea_ref, w_ref, *t_refs):
    acc = jnp.dot(fea_ref[...], w_ref[0], preferred_element_type=jnp.float32)
    for k in range(KC):
        t_refs[k][...] = acc[:, k * CW:(k + 1) * CW]


def _matmul(fea, w_stack):
    nblk = (2 * N_NODE) // BM
    half = N_NODE // BM
    out_spec = pl.BlockSpec((BM, CW), lambda i: (i, 0))
    return pl.pallas_call(
        _mm_body,
        grid=(nblk,),
        in_specs=[
            pl.BlockSpec((BM, FT), lambda i: (i, 0)),
            pl.BlockSpec((1, FT, OUT), lambda i: (i // half, 0, 0)),
        ],
        out_specs=[out_spec] * KC,
        out_shape=[jax.ShapeDtypeStruct((2 * N_NODE, CW), jnp.float32)] * KC,
    )(fea, w_stack)


def _seg_body(*refs):
    tables = refs[:KC]
    (gidx, sidx, zacc, ones_hbm, msg, deg,
     gi_v, si_v, buf0, buf1, ones_v, acc_sh, sem0, sem1) = refs[KC:]
    c = lax.axis_index("c")
    s = lax.axis_index("s")
    pltpu.sync_copy(gidx.at[c, s], gi_v)
    pltpu.sync_copy(sidx.at[c, s], si_v)
    pltpu.sync_copy(ones_hbm, ones_v)
    zb = s * ZR
    for k in range(KC + 1):
        pltpu.sync_copy(zacc.at[pl.ds(zb, ZR)], acc_sh.at[pl.ds(zb, ZR)])
        plsc.subcore_barrier()

        if k < KC:
            tk = tables[k]
            pltpu.async_copy(tk.at[gi_v.at[0]], buf0, sem0)

            def body2(j, carry):
                i0 = 2 * j
                pltpu.async_copy(tk.at[gi_v.at[i0 + 1]], buf1, sem1)
                pltpu.make_async_copy(tk.at[gi_v.at[i0]], buf0, sem0).wait()
                pltpu.sync_copy(buf0, acc_sh.at[si_v.at[i0]], add=True)
                pltpu.async_copy(tk.at[gi_v.at[i0 + 2]], buf0, sem0)
                pltpu.make_async_copy(tk.at[gi_v.at[i0 + 1]], buf1,
                                      sem1).wait()
                pltpu.sync_copy(buf1, acc_sh.at[si_v.at[i0 + 1]], add=True)
                return carry

            lax.fori_loop(0, NB // 2, body2, 0)
            pltpu.make_async_copy(tk.at[gi_v.at[NB]], buf0, sem0).wait()
        else:
            def dbody(i, carry):
                pltpu.sync_copy(ones_v, acc_sh.at[si_v.at[i]], add=True)
                return carry

            lax.fori_loop(0, NB, dbody, 0)
        plsc.subcore_barrier()

        if k < KC:
            pltpu.sync_copy(acc_sh.at[pl.ds(zb, ZR)],
                            msg.at[c, k, pl.ds(zb, ZR)])
        else:
            pltpu.sync_copy(acc_sh.at[pl.ds(zb, ZR)],
                            deg.at[c, pl.ds(zb, ZR)])
        plsc.subcore_barrier()


_seg_sum = functools.partial(
    pl.kernel,
    out_type=[
        jax.ShapeDtypeStruct((NC, KC, R_ACC, CW), jnp.float32),
        jax.ShapeDtypeStruct((NC, R_ACC, CW), jnp.float32),
    ],
    mesh=plsc.VectorSubcoreMesh(core_axis_name="c", subcore_axis_name="s"),
    compiler_params=pltpu.CompilerParams(use_tc_tiling_on_sc=False),
    scratch_types=[
        pltpu.VMEM((NBA, EB), jnp.int32),
        pltpu.VMEM((NBA, EB), jnp.int32),
        pltpu.VMEM((EB, CW), jnp.float32),
        pltpu.VMEM((EB, CW), jnp.float32),
        pltpu.VMEM((EB, CW), jnp.float32),
        pltpu.VMEM_SHARED((R_ACC, CW), jnp.float32),
        pltpu.SemaphoreType.DMA,
        pltpu.SemaphoreType.DMA,
    ],
)(_seg_body)


def _epi_body(msg_ref, deg_ref, b_ref, out_ref):
    d = jnp.maximum(deg_ref[0, :, 0], 1.0)[:, None]
    for k in range(KC):
        x = msg_ref[0, k] / d + b_ref[0, 0, k * CW:(k + 1) * CW]
        out_ref[:, k * CW:(k + 1) * CW] = jnp.maximum(x, 0.0)


def _epilogue(msg, deg, b_stack):
    half = N_NODE // BM
    return pl.pallas_call(
        _epi_body,
        grid=(NC, half),
        in_specs=[
            pl.BlockSpec((1, KC, BM, CW), lambda c, i: (c, 0, i, 0)),
            pl.BlockSpec((1, BM, CW), lambda c, i: (c, i, 0)),
            pl.BlockSpec((1, 1, OUT), lambda c, i: (c, 0, 0)),
        ],
        out_specs=pl.BlockSpec((BM, OUT), lambda c, i: (c * half + i, 0)),
        out_shape=jax.ShapeDtypeStruct((2 * N_NODE, OUT), jnp.float32),
    )(msg, deg, b_stack)


def _pack_idx(active, fill):
    a = active.reshape(NS, NB, EB)
    f = jnp.broadcast_to(fill.reshape(1, NBA - NB, EB), (NS, NBA - NB, EB))
    return jnp.concatenate([a, f], axis=1)


def kernel(ufea, vfea, Wu, Wv, bu, bv, attW_a, attb_a, attq_a,
           attW_b, attb_b, attq_b, edge_index):
    src = edge_index[0].astype(jnp.int32)
    dst = edge_index[1].astype(jnp.int32)

    fea = jnp.concatenate([vfea, ufea], axis=0)
    w_stack = jnp.stack([Wv, Wu])
    tables = _matmul(fea, w_stack)

    npad = E_PAD - N_EDGE
    pj = jnp.arange(npad, dtype=jnp.int32)
    fill = jnp.arange((NBA - NB) * EB, dtype=jnp.int32)
    g0 = jnp.concatenate([dst, pj % N_NODE])
    g1 = jnp.concatenate([src + N_NODE, pj % N_NODE + N_NODE])
    s0 = jnp.concatenate([src, N_NODE + pj % GR])
    s1 = jnp.concatenate([dst, N_NODE + pj % GR])
    gidx = jnp.stack([_pack_idx(g0, fill % N_NODE),
                      _pack_idx(g1, fill % N_NODE + N_NODE)])
    sidx = jnp.stack([_pack_idx(s0, N_NODE + fill % GR),
                      _pack_idx(s1, N_NODE + fill % GR)])

    zacc = jnp.zeros((R_ACC, CW), jnp.float32)
    ones = jnp.ones((EB, CW), jnp.float32)

    msg, deg = _seg_sum(*tables, gidx, sidx, zacc, ones)

    return _epilogue(msg, deg, jnp.stack([bu, bv]).reshape(NC, 1, OUT))


# speedup vs baseline: 6.0103x; 1.5128x over previous
import functools

import jax
import jax.numpy as jnp
from jax import lax
from jax.experimental import pallas as pl
from jax.experimental.pallas import tpu as pltpu
from jax.experimental.pallas import tpu_sc as plsc

N_NODE = 10000
N_EDGE = 160000
FT = 256
OUT = 512
NC = 2
NS = 16
KC = 8
CW = OUT // KC
EB = 512
EPT = 10240
NB = EPT // EB
NBA = 24
E_PAD = NS * EPT
GR = 240
R_ACC = N_NODE + GR
ZR = R_ACC // NS
OR = N_NODE // NS
BM = 1000


def _mm_body(fea_ref, w_ref, *t_refs):
    acc = jnp.dot(fea_ref[...], w_ref[0], preferred_element_type=jnp.float32)
    acch = acc.astype(jnp.bfloat16)
    for k in range(KC):
        t_refs[k][...] = acch[:, k * CW:(k + 1) * CW]


def _matmul(fea, w_stack):
    nblk = (2 * N_NODE) // BM
    half = N_NODE // BM
    out_spec = pl.BlockSpec((BM, CW), lambda i: (i, 0))
    return pl.pallas_call(
        _mm_body,
        grid=(nblk,),
        in_specs=[
            pl.BlockSpec((BM, FT), lambda i: (i, 0)),
            pl.BlockSpec((1, FT, OUT), lambda i: (i // half, 0, 0)),
        ],
        out_specs=[out_spec] * KC,
        out_shape=[jax.ShapeDtypeStruct((2 * N_NODE, CW), jnp.bfloat16)] * KC,
    )(fea, w_stack)


def _seg_body(*refs):
    tables = refs[:KC]
    (gidx, sidx, zacc, ones_hbm, msg, deg,
     gi_v, si_v, buf0, buf1, ones_v, acc_sh, sem0, sem1) = refs[KC:]
    c = lax.axis_index("c")
    s = lax.axis_index("s")
    pltpu.sync_copy(gidx.at[c, s], gi_v)
    pltpu.sync_copy(sidx.at[c, s], si_v)
    pltpu.sync_copy(ones_hbm, ones_v)
    zb = s * ZR
    for k in range(KC + 1):
        pltpu.sync_copy(zacc.at[pl.ds(zb, ZR)], acc_sh.at[pl.ds(zb, ZR)])
        plsc.subcore_barrier()

        if k < KC:
            tk = tables[k]
            pltpu.async_copy(tk.at[gi_v.at[0]], buf0, sem0)

            def body2(j, carry):
                i0 = 2 * j
                pltpu.async_copy(tk.at[gi_v.at[i0 + 1]], buf1, sem1)
                pltpu.make_async_copy(tk.at[gi_v.at[i0]], buf0, sem0).wait()
                pltpu.sync_copy(buf0, acc_sh.at[si_v.at[i0]], add=True)
                pltpu.async_copy(tk.at[gi_v.at[i0 + 2]], buf0, sem0)
                pltpu.make_async_copy(tk.at[gi_v.at[i0 + 1]], buf1,
                                      sem1).wait()
                pltpu.sync_copy(buf1, acc_sh.at[si_v.at[i0 + 1]], add=True)
                return carry

            lax.fori_loop(0, NB // 2, body2, 0)
            pltpu.make_async_copy(tk.at[gi_v.at[NB]], buf0, sem0).wait()
        else:
            def dbody(i, carry):
                pltpu.sync_copy(ones_v, acc_sh.at[si_v.at[i]], add=True)
                return carry

            lax.fori_loop(0, NB, dbody, 0)
        plsc.subcore_barrier()

        if k < KC:
            pltpu.sync_copy(acc_sh.at[pl.ds(zb, ZR)],
                            msg.at[c, k, pl.ds(zb, ZR)])
        else:
            pltpu.sync_copy(acc_sh.at[pl.ds(zb, ZR)],
                            deg.at[c, pl.ds(zb, ZR)])
        plsc.subcore_barrier()


_seg_sum = functools.partial(
    pl.kernel,
    out_type=[
        jax.ShapeDtypeStruct((NC, KC, R_ACC, CW), jnp.bfloat16),
        jax.ShapeDtypeStruct((NC, R_ACC, CW), jnp.bfloat16),
    ],
    mesh=plsc.VectorSubcoreMesh(core_axis_name="c", subcore_axis_name="s"),
    compiler_params=pltpu.CompilerParams(use_tc_tiling_on_sc=False),
    scratch_types=[
        pltpu.VMEM((NBA, EB), jnp.int32),
        pltpu.VMEM((NBA, EB), jnp.int32),
        pltpu.VMEM((EB, CW), jnp.bfloat16),
        pltpu.VMEM((EB, CW), jnp.bfloat16),
        pltpu.VMEM((EB, CW), jnp.bfloat16),
        pltpu.VMEM_SHARED((R_ACC, CW), jnp.bfloat16),
        pltpu.SemaphoreType.DMA,
        pltpu.SemaphoreType.DMA,
    ],
)(_seg_body)


def _epi_body(msg_ref, deg_ref, b_ref, out_ref):
    d = jnp.maximum(deg_ref[0, :, 0].astype(jnp.float32), 1.0)[:, None]
    for k in range(KC):
        m = msg_ref[0, k].astype(jnp.float32)
        x = m / d + b_ref[0, 0, k * CW:(k + 1) * CW]
        out_ref[:, k * CW:(k + 1) * CW] = jnp.maximum(x, 0.0)


def _epilogue(msg, deg, b_stack):
    half = N_NODE // BM
    return pl.pallas_call(
        _epi_body,
        grid=(NC, half),
        in_specs=[
            pl.BlockSpec((1, KC, BM, CW), lambda c, i: (c, 0, i, 0)),
            pl.BlockSpec((1, BM, CW), lambda c, i: (c, i, 0)),
            pl.BlockSpec((1, 1, OUT), lambda c, i: (c, 0, 0)),
        ],
        out_specs=pl.BlockSpec((BM, OUT), lambda c, i: (c * half + i, 0)),
        out_shape=jax.ShapeDtypeStruct((2 * N_NODE, OUT), jnp.float32),
    )(msg, deg, b_stack)


def _pack_idx(active, fill):
    a = active.reshape(NS, NB, EB)
    f = jnp.broadcast_to(fill.reshape(1, NBA - NB, EB), (NS, NBA - NB, EB))
    return jnp.concatenate([a, f], axis=1)


def kernel(ufea, vfea, Wu, Wv, bu, bv, attW_a, attb_a, attq_a,
           attW_b, attb_b, attq_b, edge_index):
    src = edge_index[0].astype(jnp.int32)
    dst = edge_index[1].astype(jnp.int32)

    fea = jnp.concatenate([vfea, ufea], axis=0)
    w_stack = jnp.stack([Wv, Wu])
    tables = _matmul(fea, w_stack)

    npad = E_PAD - N_EDGE
    pj = jnp.arange(npad, dtype=jnp.int32)
    fill = jnp.arange((NBA - NB) * EB, dtype=jnp.int32)
    g0 = jnp.concatenate([dst, pj % N_NODE])
    g1 = jnp.concatenate([src + N_NODE, pj % N_NODE + N_NODE])
    s0 = jnp.concatenate([src, N_NODE + pj % GR])
    s1 = jnp.concatenate([dst, N_NODE + pj % GR])
    gidx = jnp.stack([_pack_idx(g0, fill % N_NODE),
                      _pack_idx(g1, fill % N_NODE + N_NODE)])
    sidx = jnp.stack([_pack_idx(s0, N_NODE + fill % GR),
                      _pack_idx(s1, N_NODE + fill % GR)])

    zacc = jnp.zeros((R_ACC, CW), jnp.bfloat16)
    ones = jnp.ones((EB, CW), jnp.bfloat16)

    msg, deg = _seg_sum(*tables, gidx, sidx, zacc, ones)

    return _epilogue(msg, deg, jnp.stack([bu, bv]).reshape(NC, 1, OUT))


# speedup vs baseline: 7.0683x; 1.1760x over previous
import functools

import jax
import jax.numpy as jnp
from jax import lax
from jax.experimental import pallas as pl
from jax.experimental.pallas import tpu as pltpu
from jax.experimental.pallas import tpu_sc as plsc

N_NODE = 10000
N_EDGE = 160000
FT = 256
OUT = 512
NC = 2
NS = 16
KC = 8
CW = OUT // KC
EB = 512
EPT = 10240
NB = EPT // EB
NBA = 24
E_PAD = NS * EPT
GR = 240
R_ACC = N_NODE + GR
ZR = R_ACC // NS
BM = 1000


def _mm_body(fea_ref, w_ref, out_ref):
    out_ref[...] = jnp.dot(fea_ref[...], w_ref[0],
                           preferred_element_type=jnp.float32)


def _matmul(fea, w_stack):
    nblk = (2 * N_NODE) // BM
    half = N_NODE // BM
    return pl.pallas_call(
        _mm_body,
        grid=(nblk,),
        in_specs=[
            pl.BlockSpec((BM, FT), lambda i: (i, 0)),
            pl.BlockSpec((1, FT, OUT), lambda i: (i // half, 0, 0)),
        ],
        out_specs=pl.BlockSpec((BM, OUT), lambda i: (i, 0)),
        out_shape=jax.ShapeDtypeStruct((2 * N_NODE, OUT), jnp.float32),
    )(fea, w_stack)


def _chunk_body(table, gidx, sidx, zacc, msg,
                gi_v, si_v, buf0, buf1, acc_sh, sem0, sem1):
    c = lax.axis_index("c")
    s = lax.axis_index("s")
    pltpu.sync_copy(gidx.at[c, s], gi_v)
    pltpu.sync_copy(sidx.at[c, s], si_v)
    zb = s * ZR
    pltpu.sync_copy(zacc.at[pl.ds(zb, ZR)], acc_sh.at[pl.ds(zb, ZR)])
    plsc.subcore_barrier()

    pltpu.async_copy(table.at[gi_v.at[0]], buf0, sem0)

    def body2(j, carry):
        i0 = 2 * j
        pltpu.async_copy(table.at[gi_v.at[i0 + 1]], buf1, sem1)
        pltpu.make_async_copy(table.at[gi_v.at[i0]], buf0, sem0).wait()
        pltpu.sync_copy(buf0, acc_sh.at[si_v.at[i0]], add=True)
        pltpu.async_copy(table.at[gi_v.at[i0 + 2]], buf0, sem0)
        pltpu.make_async_copy(table.at[gi_v.at[i0 + 1]], buf1, sem1).wait()
        pltpu.sync_copy(buf1, acc_sh.at[si_v.at[i0 + 1]], add=True)
        return carry

    lax.fori_loop(0, NB // 2, body2, 0)
    pltpu.make_async_copy(table.at[gi_v.at[NB]], buf0, sem0).wait()
    plsc.subcore_barrier()
    pltpu.sync_copy(acc_sh.at[pl.ds(zb, ZR)], msg.at[c, pl.ds(zb, ZR)])


_seg_chunk = functools.partial(
    pl.kernel,
    out_type=jax.ShapeDtypeStruct((NC, R_ACC, CW), jnp.bfloat16),
    mesh=plsc.VectorSubcoreMesh(core_axis_name="c", subcore_axis_name="s"),
    compiler_params=pltpu.CompilerParams(use_tc_tiling_on_sc=False),
    scratch_types=[
        pltpu.VMEM((NBA, EB), jnp.int32),
        pltpu.VMEM((NBA, EB), jnp.int32),
        pltpu.VMEM((EB, CW), jnp.bfloat16),
        pltpu.VMEM((EB, CW), jnp.bfloat16),
        pltpu.VMEM_SHARED((R_ACC, CW), jnp.bfloat16),
        pltpu.SemaphoreType.DMA,
        pltpu.SemaphoreType.DMA,
    ],
)(_chunk_body)


def _deg_body(sidx, zacc, ones_hbm, deg,
              si_v, ones_v, acc_sh, sem0):
    c = lax.axis_index("c")
    s = lax.axis_index("s")
    pltpu.sync_copy(sidx.at[c, s], si_v)
    pltpu.sync_copy(ones_hbm, ones_v)
    zb = s * ZR
    pltpu.sync_copy(zacc.at[pl.ds(zb, ZR)], acc_sh.at[pl.ds(zb, ZR)])
    plsc.subcore_barrier()

    def dbody(i, carry):
        pltpu.sync_copy(ones_v, acc_sh.at[si_v.at[i]], add=True)
        return carry

    lax.fori_loop(0, NB, dbody, 0)
    plsc.subcore_barrier()
    pltpu.sync_copy(acc_sh.at[pl.ds(zb, ZR)], deg.at[c, pl.ds(zb, ZR)])


_seg_deg = functools.partial(
    pl.kernel,
    out_type=jax.ShapeDtypeStruct((NC, R_ACC, CW), jnp.bfloat16),
    mesh=plsc.VectorSubcoreMesh(core_axis_name="c", subcore_axis_name="s"),
    compiler_params=pltpu.CompilerParams(use_tc_tiling_on_sc=False),
    scratch_types=[
        pltpu.VMEM((NBA, EB), jnp.int32),
        pltpu.VMEM((EB, CW), jnp.bfloat16),
        pltpu.VMEM_SHARED((R_ACC, CW), jnp.bfloat16),
        pltpu.SemaphoreType.DMA,
    ],
)(_deg_body)


def _epi_body(*refs):
    msg_refs = refs[:KC]
    deg_ref, b_ref, out_ref = refs[KC:]
    d = jnp.maximum(deg_ref[0, :, 0].astype(jnp.float32), 1.0)[:, None]
    for k in range(KC):
        m = msg_refs[k][0].astype(jnp.float32)
        x = m / d + b_ref[0, 0, k * CW:(k + 1) * CW]
        out_ref[:, k * CW:(k + 1) * CW] = jnp.maximum(x, 0.0)


def _epilogue(msgs, deg, b_stack):
    half = N_NODE // BM
    m_spec = pl.BlockSpec((1, BM, CW), lambda c, i: (c, i, 0))
    return pl.pallas_call(
        _epi_body,
        grid=(NC, half),
        in_specs=[m_spec] * KC + [
            m_spec,
            pl.BlockSpec((1, 1, OUT), lambda c, i: (c, 0, 0)),
        ],
        out_specs=pl.BlockSpec((BM, OUT), lambda c, i: (c * half + i, 0)),
        out_shape=jax.ShapeDtypeStruct((2 * N_NODE, OUT), jnp.float32),
    )(*msgs, deg, b_stack)


def _pack_idx(active, fill):
    a = active.reshape(NS, NB, EB)
    f = jnp.broadcast_to(fill.reshape(1, NBA - NB, EB), (NS, NBA - NB, EB))
    return jnp.concatenate([a, f], axis=1).reshape(1, NS, NBA, EB)


def kernel(ufea, vfea, Wu, Wv, bu, bv, attW_a, attb_a, attq_a,
           attW_b, attb_b, attq_b, edge_index):
    src = edge_index[0].astype(jnp.int32)
    dst = edge_index[1].astype(jnp.int32)

    fea = jnp.concatenate([vfea, ufea], axis=0)
    w_stack = jnp.stack([Wv, Wu])
    h = _matmul(fea, w_stack)
    tables = [h[:, k * CW:(k + 1) * CW].astype(jnp.bfloat16)
              for k in range(KC)]

    npad = E_PAD - N_EDGE
    pj = jnp.arange(npad, dtype=jnp.int32)
    fill = jnp.arange((NBA - NB) * EB, dtype=jnp.int32)
    g0 = jnp.concatenate([dst, pj % N_NODE])
    g1 = jnp.concatenate([src + N_NODE, pj % N_NODE + N_NODE])
    s0 = jnp.concatenate([src, N_NODE + pj % GR])
    s1 = jnp.concatenate([dst, N_NODE + pj % GR])
    gidx = jnp.concatenate([_pack_idx(g0, fill % N_NODE),
                            _pack_idx(g1, fill % N_NODE + N_NODE)])
    sidx = jnp.concatenate([_pack_idx(s0, N_NODE + fill % GR),
                            _pack_idx(s1, N_NODE + fill % GR)])

    zacc = jnp.zeros((R_ACC, CW), jnp.bfloat16)
    ones = jnp.ones((EB, CW), jnp.bfloat16)

    deg = _seg_deg(sidx, zacc, ones)
    msgs = [_seg_chunk(tables[k], gidx, sidx, zacc) for k in range(KC)]

    return _epilogue(msgs, deg, jnp.stack([bu, bv]).reshape(NC, 1, OUT))


# speedup vs baseline: 7.1087x; 1.0057x over previous
import functools

import jax
import jax.numpy as jnp
from jax import lax
from jax.experimental import pallas as pl
from jax.experimental.pallas import tpu as pltpu
from jax.experimental.pallas import tpu_sc as plsc

N_NODE = 10000
N_EDGE = 160000
FT = 256
OUT = 512
NC = 2
NS = 16
KC = 8
CW = OUT // KC
EB = 512
EPT = 10240
NB = EPT // EB
NBA = 24
E_PAD = NS * EPT
GR = 240
R_ACC = N_NODE + GR
ZR = R_ACC // NS
BM = 1000


def _mm_body(fea_ref, w_ref, out_ref):
    out_ref[...] = jnp.dot(fea_ref[...], w_ref[0],
                           preferred_element_type=jnp.float32)


def _matmul(fea, w_stack):
    nblk = (2 * N_NODE) // BM
    half = N_NODE // BM
    return pl.pallas_call(
        _mm_body,
        grid=(nblk,),
        in_specs=[
            pl.BlockSpec((BM, FT), lambda i: (i, 0)),
            pl.BlockSpec((1, FT, OUT), lambda i: (i // half, 0, 0)),
        ],
        out_specs=pl.BlockSpec((BM, OUT), lambda i: (i, 0)),
        out_shape=jax.ShapeDtypeStruct((2 * N_NODE, OUT), jnp.float32),
    )(fea, w_stack)


def _chunk_body(table, gidx, sidx, zacc, msg,
                gi_v, si_v, buf0, buf1, acc_sh, sem0, sem1):
    c = lax.axis_index("c")
    s = lax.axis_index("s")
    pltpu.sync_copy(gidx.at[c, s], gi_v)
    pltpu.sync_copy(sidx.at[c, s], si_v)
    zb = s * ZR
    pltpu.sync_copy(zacc.at[pl.ds(zb, ZR)], acc_sh.at[pl.ds(zb, ZR)])
    plsc.subcore_barrier()

    pltpu.async_copy(table.at[gi_v.at[0]], buf0, sem0)

    def body2(j, carry):
        i0 = 2 * j
        pltpu.async_copy(table.at[gi_v.at[i0 + 1]], buf1, sem1)
        pltpu.make_async_copy(table.at[gi_v.at[i0]], buf0, sem0).wait()
        pltpu.sync_copy(buf0, acc_sh.at[si_v.at[i0]], add=True)
        pltpu.async_copy(table.at[gi_v.at[i0 + 2]], buf0, sem0)
        pltpu.make_async_copy(table.at[gi_v.at[i0 + 1]], buf1, sem1).wait()
        pltpu.sync_copy(buf1, acc_sh.at[si_v.at[i0 + 1]], add=True)
        return carry

    lax.fori_loop(0, NB // 2, body2, 0)
    pltpu.make_async_copy(table.at[gi_v.at[NB]], buf0, sem0).wait()
    plsc.subcore_barrier()
    pltpu.sync_copy(acc_sh.at[pl.ds(zb, ZR)], msg.at[c, pl.ds(zb, ZR)])


_seg_chunk = functools.partial(
    pl.kernel,
    out_type=jax.ShapeDtypeStruct((NC, R_ACC, CW), jnp.bfloat16),
    mesh=plsc.VectorSubcoreMesh(core_axis_name="c", subcore_axis_name="s"),
    compiler_params=pltpu.CompilerParams(use_tc_tiling_on_sc=False),
    scratch_types=[
        pltpu.VMEM((NBA, EB), jnp.int32),
        pltpu.VMEM((NBA, EB), jnp.int32),
        pltpu.VMEM((EB, CW), jnp.bfloat16),
        pltpu.VMEM((EB, CW), jnp.bfloat16),
        pltpu.VMEM_SHARED((R_ACC, CW), jnp.bfloat16),
        pltpu.SemaphoreType.DMA,
        pltpu.SemaphoreType.DMA,
    ],
)(_chunk_body)


def _deg_body(sidx, zacc, ones_hbm, deg,
              si_v, ones_v, acc_sh, sem0):
    c = lax.axis_index("c")
    s = lax.axis_index("s")
    pltpu.sync_copy(sidx.at[c, s], si_v)
    pltpu.sync_copy(ones_hbm, ones_v)
    zb = s * ZR
    pltpu.sync_copy(zacc.at[pl.ds(zb, ZR)], acc_sh.at[pl.ds(zb, ZR)])
    plsc.subcore_barrier()

    def dbody(i, carry):
        pltpu.sync_copy(ones_v, acc_sh.at[si_v.at[i]], add=True)
        return carry

    lax.fori_loop(0, NB, dbody, 0)
    plsc.subcore_barrier()
    pltpu.sync_copy(acc_sh.at[pl.ds(zb, ZR)], deg.at[c, pl.ds(zb, ZR)])


DW = 32

_seg_deg = functools.partial(
    pl.kernel,
    out_type=jax.ShapeDtypeStruct((NC, R_ACC, DW), jnp.bfloat16),
    mesh=plsc.VectorSubcoreMesh(core_axis_name="c", subcore_axis_name="s"),
    compiler_params=pltpu.CompilerParams(use_tc_tiling_on_sc=False),
    scratch_types=[
        pltpu.VMEM((NBA, EB), jnp.int32),
        pltpu.VMEM((EB, DW), jnp.bfloat16),
        pltpu.VMEM_SHARED((R_ACC, DW), jnp.bfloat16),
        pltpu.SemaphoreType.DMA,
    ],
)(_deg_body)


def _epi_body(*refs):
    msg_refs = refs[:KC]
    deg_ref, b_ref, out_ref = refs[KC:]
    d = jnp.maximum(deg_ref[0, :, 0].astype(jnp.float32), 1.0)[:, None]
    for k in range(KC):
        m = msg_refs[k][0].astype(jnp.float32)
        x = m / d + b_ref[0, 0, k * CW:(k + 1) * CW]
        out_ref[:, k * CW:(k + 1) * CW] = jnp.maximum(x, 0.0)


def _epilogue(msgs, deg, b_stack):
    half = N_NODE // BM
    m_spec = pl.BlockSpec((1, BM, CW), lambda c, i: (c, i, 0))
    return pl.pallas_call(
        _epi_body,
        grid=(NC, half),
        in_specs=[m_spec] * KC + [
            pl.BlockSpec((1, BM, DW), lambda c, i: (c, i, 0)),
            pl.BlockSpec((1, 1, OUT), lambda c, i: (c, 0, 0)),
        ],
        out_specs=pl.BlockSpec((BM, OUT), lambda c, i: (c * half + i, 0)),
        out_shape=jax.ShapeDtypeStruct((2 * N_NODE, OUT), jnp.float32),
    )(*msgs, deg, b_stack)


def _pack_idx(active, fill):
    a = active.reshape(NS, NB, EB)
    f = jnp.broadcast_to(fill.reshape(1, NBA - NB, EB), (NS, NBA - NB, EB))
    return jnp.concatenate([a, f], axis=1).reshape(1, NS, NBA, EB)


def kernel(ufea, vfea, Wu, Wv, bu, bv, attW_a, attb_a, attq_a,
           attW_b, attb_b, attq_b, edge_index):
    src = edge_index[0].astype(jnp.int32)
    dst = edge_index[1].astype(jnp.int32)

    fea = jnp.concatenate([vfea, ufea], axis=0)
    w_stack = jnp.stack([Wv, Wu])
    h = _matmul(fea, w_stack)
    tables = [h[:, k * CW:(k + 1) * CW].astype(jnp.bfloat16)
              for k in range(KC)]

    npad = E_PAD - N_EDGE
    pj = jnp.arange(npad, dtype=jnp.int32)
    fill = jnp.arange((NBA - NB) * EB, dtype=jnp.int32)
    g0 = jnp.concatenate([dst, pj % N_NODE])
    g1 = jnp.concatenate([src + N_NODE, pj % N_NODE + N_NODE])
    s0 = jnp.concatenate([src, N_NODE + pj % GR])
    s1 = jnp.concatenate([dst, N_NODE + pj % GR])
    gidx = jnp.concatenate([_pack_idx(g0, fill % N_NODE),
                            _pack_idx(g1, fill % N_NODE + N_NODE)])
    sidx = jnp.concatenate([_pack_idx(s0, N_NODE + fill % GR),
                            _pack_idx(s1, N_NODE + fill % GR)])

    zacc = jnp.zeros((R_ACC, CW), jnp.bfloat16)
    zdeg = jnp.zeros((R_ACC, DW), jnp.bfloat16)
    ones = jnp.ones((EB, DW), jnp.bfloat16)

    deg = _seg_deg(sidx, zdeg, ones)
    msgs = [_seg_chunk(tables[k], gidx, sidx, zacc) for k in range(KC)]

    return _epilogue(msgs, deg, jnp.stack([bu, bv]).reshape(NC, 1, OUT))
